# Initial kernel scaffold; baseline (speedup 1.0000x reference)
import jax
import jax.numpy as jnp
from jax.experimental import pallas as pl


def kernel(logits):
    raise NotImplementedError("write your pallas kernel here")


# speedup vs baseline: 22.6386x; 22.6386x over previous
import functools

import jax
import jax.numpy as jnp
from jax import lax
from jax.experimental import pallas as pl
from jax.experimental.pallas import tpu as pltpu
from jax.experimental.pallas import tpu_sc as plsc

ROWS = 64
N = 100000
L = 16
VREGS = N // L
NB = 512
NCHUNK = NB // L
P_TOP = 0.5

_mesh = plsc.VectorSubcoreMesh(
    core_axis_name="c", subcore_axis_name="s", num_cores=2, num_subcores=16
)


@functools.partial(
    pl.kernel,
    out_type=jax.ShapeDtypeStruct((ROWS, N), jnp.float32),
    mesh=_mesh,
    scratch_types=[
        pltpu.VMEM((N,), jnp.float32),
        pltpu.VMEM((NB * L,), jnp.float32),
        pltpu.VMEM((NB + L,), jnp.float32),
    ],
    compiler_params=pltpu.CompilerParams(needs_layout_passes=False),
)
def _nucleus_kernel(logits_hbm, out_hbm, row_v, hist_v, sbuf_v):
    wid = lax.axis_index("s") * 2 + lax.axis_index("c")
    lanes = lax.iota(jnp.int32, L)
    lane_base = lanes * NB
    zf = jnp.zeros((L,), jnp.float32)

    def run_level(shift, prefix, a_above, p_mass):
        def zero_body(i, _):
            hist_v[pl.ds(i * L, L)] = zf
            return 0
        lax.fori_loop(0, NB * L // L, zero_body, 0)

        def scan_body(i, _):
            e = row_v[pl.ds(i * L, L)]
            key = plsc.bitcast(e, jnp.int32)
            sel = lax.shift_right_logical(key, shift + 9) == prefix
            b = lax.shift_right_logical(key, shift) & (NB - 1)
            plsc.addupdate_scatter(hist_v, [lane_base + b], e, mask=sel)
            return 0
        lax.fori_loop(0, VREGS, scan_body, 0)

        sbuf_v[pl.ds(NB, L)] = a_above

        def suffix_body(k, carry):
            carry_v, cnt = carry
            cc = NCHUNK - 1 - k
            tot = zf
            for l in range(L):
                tot = tot + hist_v[pl.ds(l * NB + cc * L, L)]
            suf = lax.rev(plsc.cumsum(lax.rev(tot, (0,))), (0,))
            s_chunk = suf + carry_v
            sbuf_v[pl.ds(cc * L, L)] = s_chunk
            cnt = cnt + plsc.all_reduce_population_count(s_chunk >= p_mass)
            return jnp.full((L,), jnp.max(s_chunk)), cnt

        _, cnt = lax.fori_loop(
            0, NCHUNK, suffix_body, (a_above, jnp.zeros((L,), jnp.int32))
        )
        b_lvl = jnp.maximum(cnt - 1, 0)
        a_next = plsc.load_gather(sbuf_v, [b_lvl + 1])
        return b_lvl, a_next

    for j in range(2):
        row = wid * 2 + j
        pltpu.sync_copy(logits_hbm.at[row], row_v)

        def exp_body(i, acc):
            e = jnp.exp(row_v[pl.ds(i * L, L)])
            row_v[pl.ds(i * L, L)] = e
            return acc + e
        acc = lax.fori_loop(0, VREGS, exp_body, zf)
        z = jnp.sum(acc)
        z_vec = jnp.full((L,), z)
        p_mass = jnp.float32(P_TOP) * z_vec
        inv_z = jnp.ones((L,), jnp.float32) / z_vec

        zi = jnp.zeros((L,), jnp.int32)
        b1, a1 = run_level(22, zi, zf, p_mass)
        b2, a2 = run_level(13, b1, a1, p_mass)
        b3, _ = run_level(4, (b1 * NB) + b2, a2, p_mass)
        kmin = ((b1 * NB) + b2) * NB + b3

        def out_body(i, _):
            e = row_v[pl.ds(i * L, L)]
            key = plsc.bitcast(e, jnp.int32)
            keep = lax.shift_right_logical(key, 4) >= kmin
            row_v[pl.ds(i * L, L)] = jnp.where(keep, e * inv_z, 0.0)
            return 0
        lax.fori_loop(0, VREGS, out_body, 0)

        pltpu.sync_copy(row_v, out_hbm.at[row])


def kernel(logits):
    return _nucleus_kernel(logits)


# speedup vs baseline: 28.5525x; 1.2612x over previous
import functools

import jax
import jax.numpy as jnp
from jax import lax
from jax.experimental import pallas as pl
from jax.experimental.pallas import tpu as pltpu
from jax.experimental.pallas import tpu_sc as plsc

ROWS = 64
N = 100000
L = 16
VREGS = N // L
NB = 512
NCHUNK = NB // L
P_TOP = 0.5
U = 10

_mesh = plsc.VectorSubcoreMesh(
    core_axis_name="c", subcore_axis_name="s", num_cores=2, num_subcores=16
)


@functools.partial(
    pl.kernel,
    out_type=jax.ShapeDtypeStruct((ROWS, N), jnp.float32),
    mesh=_mesh,
    scratch_types=[
        pltpu.VMEM((N,), jnp.float32),
        pltpu.VMEM((NB * L,), jnp.float32),
        pltpu.VMEM((NB + L,), jnp.float32),
    ],
    compiler_params=pltpu.CompilerParams(needs_layout_passes=False),
)
def _nucleus_kernel(logits_hbm, out_hbm, row_v, hist_v, sbuf_v):
    wid = lax.axis_index("s") * 2 + lax.axis_index("c")
    lanes = lax.iota(jnp.int32, L)
    lane_base = lanes * NB
    zf = jnp.zeros((L,), jnp.float32)

    def zero_hist():
        def zero_body(i, _):
            for u in range(8):
                hist_v[pl.ds((i * 8 + u) * L, L)] = zf
            return 0
        lax.fori_loop(0, NB // 8, zero_body, 0)

    def suffix_scan(a_above, p_mass):
        sbuf_v[pl.ds(NB, L)] = a_above

        def suffix_body(k, carry):
            carry_v, cnt = carry
            cc = NCHUNK - 1 - k
            tot = zf
            for l in range(L):
                tot = tot + hist_v[pl.ds(l * NB + cc * L, L)]
            suf = lax.rev(plsc.cumsum(lax.rev(tot, (0,))), (0,))
            s_chunk = suf + carry_v
            sbuf_v[pl.ds(cc * L, L)] = s_chunk
            cnt = cnt + plsc.all_reduce_population_count(s_chunk >= p_mass)
            return jnp.full((L,), jnp.max(s_chunk)), cnt

        _, cnt = lax.fori_loop(
            0, NCHUNK, suffix_body, (a_above, jnp.zeros((L,), jnp.int32))
        )
        b_lvl = jnp.maximum(cnt - 1, 0)
        a_next = plsc.load_gather(sbuf_v, [b_lvl + 1])
        return b_lvl, a_next

    def run_level(shift, prefix, a_above, p_mass):
        zero_hist()

        def scan_body(i, _):
            for u in range(U):
                e = row_v[pl.ds((i * U + u) * L, L)]
                key = plsc.bitcast(e, jnp.int32)
                sel = lax.shift_right_logical(key, shift + 9) == prefix
                b = lax.shift_right_logical(key, shift) & (NB - 1)
                plsc.addupdate_scatter(hist_v, [lane_base + b], e, mask=sel)
            return 0
        lax.fori_loop(0, VREGS // U, scan_body, 0)
        return suffix_scan(a_above, p_mass)

    for j in range(2):
        row = wid * 2 + j
        pltpu.sync_copy(logits_hbm.at[row], row_v)
        zero_hist()

        def exp_body(i, acc):
            part = None
            for u in range(U):
                e = jnp.exp(row_v[pl.ds((i * U + u) * L, L)])
                row_v[pl.ds((i * U + u) * L, L)] = e
                key = plsc.bitcast(e, jnp.int32)
                b = lax.shift_right_logical(key, 22)
                plsc.addupdate_scatter(hist_v, [lane_base + b], e)
                part = e if part is None else part + e
            return acc + part
        acc = lax.fori_loop(0, VREGS // U, exp_body, zf)
        z = jnp.sum(acc)
        z_vec = jnp.full((L,), z)
        p_mass = jnp.float32(P_TOP) * z_vec
        inv_z = jnp.ones((L,), jnp.float32) / z_vec

        b1, a1 = suffix_scan(zf, p_mass)
        b2, a2 = run_level(13, b1, a1, p_mass)
        b3, _ = run_level(4, (b1 * NB) + b2, a2, p_mass)
        kmin = ((b1 * NB) + b2) * NB + b3

        def out_body(i, _):
            for u in range(U):
                e = row_v[pl.ds((i * U + u) * L, L)]
                key = plsc.bitcast(e, jnp.int32)
                keep = lax.shift_right_logical(key, 4) >= kmin
                row_v[pl.ds((i * U + u) * L, L)] = jnp.where(keep, e * inv_z, 0.0)
            return 0
        lax.fori_loop(0, VREGS // U, out_body, 0)

        pltpu.sync_copy(row_v, out_hbm.at[row])


def kernel(logits):
    return _nucleus_kernel(logits)


# speedup vs baseline: 62.5770x; 2.1916x over previous
import functools

import jax
import jax.numpy as jnp
from jax import lax
from jax.experimental import pallas as pl
from jax.experimental.pallas import tpu as pltpu
from jax.experimental.pallas import tpu_sc as plsc

ROWS = 64
N = 100000
L = 16
VREGS = N // L
NB = 512
NCHUNK = NB // L
P_TOP = 0.5
C = 10

LOG2E = 1.4426950408889634
MAGIC = 12582912.0
LN2_HI = 0.693359375
LN2_LO = -2.12194440e-4
EC = (1.9875691500e-4, 1.3981999507e-3, 8.3334519073e-3,
      4.1665795894e-2, 1.6666665459e-1, 5.0000001201e-1)

_mesh = plsc.VectorSubcoreMesh(
    core_axis_name="c", subcore_axis_name="s", num_cores=2, num_subcores=16
)


def _poly_exp(xs):
    f = jnp.float32
    zs = [x * f(LOG2E) for x in xs]
    nfs = [z + f(MAGIC) for z in zs]
    nfs = [nf - f(MAGIC) for nf in nfs]
    nis = [nf.astype(jnp.int32) for nf in nfs]
    rs = [x - nf * f(LN2_HI) for x, nf in zip(xs, nfs)]
    rs = [r - nf * f(LN2_LO) for r, nf in zip(rs, nfs)]
    ps = [f(EC[0]) * r + f(EC[1]) for r in rs]
    for c in EC[2:]:
        ps = [p * r + f(c) for p, r in zip(ps, rs)]
    r2s = [r * r for r in rs]
    ers = [p * r2 + r for p, r2, r in zip(ps, r2s, rs)]
    ers = [er + f(1.0) for er in ers]
    scs = [(ni + 127) << 23 for ni in nis]
    scs = [plsc.bitcast(sc, jnp.float32) for sc in scs]
    return [er * sc for er, sc in zip(ers, scs)]


@functools.partial(
    pl.kernel,
    out_type=jax.ShapeDtypeStruct((ROWS, N), jnp.float32),
    mesh=_mesh,
    scratch_types=[
        pltpu.VMEM((N,), jnp.float32),
        pltpu.VMEM((NB * L,), jnp.float32),
        pltpu.VMEM((NB + L,), jnp.float32),
    ],
    compiler_params=pltpu.CompilerParams(needs_layout_passes=False),
)
def _nucleus_kernel(logits_hbm, out_hbm, row_v, hist_v, sbuf_v):
    wid = lax.axis_index("s") * 2 + lax.axis_index("c")
    lanes = lax.iota(jnp.int32, L)
    lane_base = lanes * NB
    zf = jnp.zeros((L,), jnp.float32)

    def zero_hist():
        def zero_body(i, _):
            for u in range(8):
                hist_v[pl.ds((i * 8 + u) * L, L)] = zf
            return 0
        lax.fori_loop(0, NB // 8, zero_body, 0)

    def suffix_scan(a_above, p_mass):
        sbuf_v[pl.ds(NB, L)] = a_above

        def suffix_body(k, carry):
            carry_v, cnt = carry
            cc = NCHUNK - 1 - k
            tot = zf
            for l in range(L):
                tot = tot + hist_v[pl.ds(l * NB + cc * L, L)]
            suf = lax.rev(plsc.cumsum(lax.rev(tot, (0,))), (0,))
            s_chunk = suf + carry_v
            sbuf_v[pl.ds(cc * L, L)] = s_chunk
            cnt = cnt + plsc.all_reduce_population_count(s_chunk >= p_mass)
            return jnp.full((L,), jnp.max(s_chunk)), cnt

        _, cnt = lax.fori_loop(
            0, NCHUNK, suffix_body, (a_above, jnp.zeros((L,), jnp.int32))
        )
        b_lvl = jnp.maximum(cnt - 1, 0)
        a_next = plsc.load_gather(sbuf_v, [b_lvl + 1])
        return b_lvl, a_next

    def run_level(shift, prefix, a_above, p_mass):
        zero_hist()

        def scan_body(i, _):
            es = [row_v[pl.ds((i * C + u) * L, L)] for u in range(C)]
            keys = [plsc.bitcast(e, jnp.int32) for e in es]
            sels = [lax.shift_right_logical(k, shift + 9) == prefix
                    for k in keys]
            bs = [lax.shift_right_logical(k, shift) & (NB - 1) for k in keys]
            idxs = [lane_base + b for b in bs]
            for u in range(C):
                plsc.addupdate_scatter(hist_v, [idxs[u]], es[u], mask=sels[u])
            return 0
        lax.fori_loop(0, VREGS // C, scan_body, 0)
        return suffix_scan(a_above, p_mass)

    for j in range(2):
        row = wid * 2 + j
        pltpu.sync_copy(logits_hbm.at[row], row_v)
        zero_hist()

        def exp_body(i, acc):
            xs = [row_v[pl.ds((i * C + u) * L, L)] for u in range(C)]
            es = _poly_exp(xs)
            for u in range(C):
                row_v[pl.ds((i * C + u) * L, L)] = es[u]
            keys = [plsc.bitcast(e, jnp.int32) for e in es]
            bs = [lax.shift_right_logical(k, 22) for k in keys]
            idxs = [lane_base + b for b in bs]
            for u in range(C):
                plsc.addupdate_scatter(hist_v, [idxs[u]], es[u])
            t = es
            while len(t) > 1:
                t = [a + b for a, b in zip(t[0::2], t[1::2])] + (
                    [t[-1]] if len(t) % 2 else [])
            return acc + t[0]
        acc = lax.fori_loop(0, VREGS // C, exp_body, zf)
        z = jnp.sum(acc)
        z_vec = jnp.full((L,), z)
        p_mass = jnp.float32(P_TOP) * z_vec
        inv_z = jnp.ones((L,), jnp.float32) / z_vec

        b1, a1 = suffix_scan(zf, p_mass)
        b2, a2 = run_level(13, b1, a1, p_mass)
        b3, _ = run_level(4, (b1 * NB) + b2, a2, p_mass)
        kmin = ((b1 * NB) + b2) * NB + b3

        def out_body(i, _):
            es = [row_v[pl.ds((i * C + u) * L, L)] for u in range(C)]
            keys = [plsc.bitcast(e, jnp.int32) for e in es]
            keeps = [lax.shift_right_logical(k, 4) >= kmin for k in keys]
            outs = [jnp.where(kp, e * inv_z, 0.0)
                    for kp, e in zip(keeps, es)]
            for u in range(C):
                row_v[pl.ds((i * C + u) * L, L)] = outs[u]
            return 0
        lax.fori_loop(0, VREGS // C, out_body, 0)

        pltpu.sync_copy(row_v, out_hbm.at[row])


def kernel(logits):
    return _nucleus_kernel(logits)
